# baseline (speedup 1.0000x reference)
import functools

import jax
import jax.numpy as jnp
from jax import lax
from jax.experimental import pallas as pl
from jax.experimental.pallas import tpu as pltpu
from jax.experimental.pallas import tpu_sc as plsc

N_NODES = 10000
N_EDGES = 320000
F_IN = 128
H = 16
C = 2

NP = 10240
EP = 327680
NC = 2
NS = 16
CH = 1024
EPT = EP // (NC * NS)
NCH = EPT // CH
NPT = NP // NS


def _sc_mesh():
    return plsc.VectorSubcoreMesh(core_axis_name="c", subcore_axis_name="s",
                                  num_cores=NC, num_subcores=NS)


_SC_PARAMS = pltpu.CompilerParams(use_tc_tiling_on_sc=False,
                                  needs_layout_passes=False)


def _rsqrt16(x):
    i = plsc.bitcast(x, jnp.int32)
    i = jnp.int32(0x5F3759DF) - lax.shift_right_logical(i, 1)
    y = plsc.bitcast(i, jnp.float32)
    hx = x * 0.5
    for _ in range(3):
        y = y * (1.5 - hx * y * y)
    return y


def _load_dinv(degp_hbm, d0_v, d1_v, s):
    pltpu.sync_copy(degp_hbm.at[0, pl.ds(s * NPT, NPT)], d0_v)
    pltpu.sync_copy(degp_hbm.at[1, pl.ds(s * NPT, NPT)], d1_v)

    def body(i, _):
        d = d0_v[pl.ds(i * 16, 16)] + d1_v[pl.ds(i * 16, 16)] + 1.0
        d0_v[pl.ds(i * 16, 16)] = _rsqrt16(d)
        return 0
    lax.fori_loop(0, NPT // 16, body, 0)


@functools.partial(
    pl.kernel,
    out_type=jax.ShapeDtypeStruct((NC, NP), jnp.float32),
    mesh=_sc_mesh(),
    compiler_params=_SC_PARAMS,
    scratch_types=[
        pltpu.VMEM((NCH, CH), jnp.int32),
        pltpu.VMEM((CH,), jnp.float32),
        pltpu.VMEM((NPT,), jnp.float32),
        pltpu.VMEM_SHARED((NP,), jnp.float32),
    ],
)
def _deg_kernel(dst_hbm, out_hbm, idx_v, ones_v, buf_v, deg_sh):
    c = lax.axis_index("c")
    s = lax.axis_index("s")
    wid = s * NC + c

    def fill(i, _):
        buf_v[pl.ds(i * 16, 16)] = jnp.zeros((16,), jnp.float32)
        return 0
    lax.fori_loop(0, NPT // 16, fill, 0)

    def fill1(i, _):
        ones_v[pl.ds(i * 16, 16)] = jnp.ones((16,), jnp.float32)
        return 0
    lax.fori_loop(0, CH // 16, fill1, 0)

    pltpu.sync_copy(buf_v, deg_sh.at[pl.ds(s * NPT, NPT)])
    for j in range(NCH):
        pltpu.sync_copy(dst_hbm.at[pl.ds(wid * EPT + j * CH, CH)], idx_v.at[j])
    plsc.subcore_barrier()

    for j in range(NCH):
        pltpu.sync_copy(ones_v, deg_sh.at[idx_v.at[j]], add=True)
    plsc.subcore_barrier()

    pltpu.sync_copy(deg_sh.at[pl.ds(s * NPT, NPT)], buf_v)
    pltpu.sync_copy(buf_v, out_hbm.at[c, pl.ds(s * NPT, NPT)])


def _load_edges(src_hbm, dst_hbm, src_v, dst_v, wid):
    for j in range(NCH):
        pltpu.sync_copy(src_hbm.at[pl.ds(wid * EPT + j * CH, CH)], src_v.at[j])
        pltpu.sync_copy(dst_hbm.at[pl.ds(wid * EPT + j * CH, CH)], dst_v.at[j])


def _edge_pass(tab_ref, src_v, dst_v, rows_v, acc_sh, sems):
    def gather(j):
        return pltpu.async_copy(tab_ref.at[src_v.at[j]], rows_v.at[j % 2],
                                sems[j % 2])

    desc = {0: gather(0)}
    for j in range(NCH):
        if j + 1 < NCH:
            desc[j + 1] = gather(j + 1)
        desc[j].wait()
        pltpu.sync_copy(rows_v.at[j % 2], acc_sh.at[dst_v.at[j]], add=True)


@functools.partial(
    pl.kernel,
    out_type=(
        jax.ShapeDtypeStruct((NC, NP, H), jnp.float32),
        jax.ShapeDtypeStruct((NC, NP, H), jnp.float32),
    ),
    mesh=_sc_mesh(),
    compiler_params=_SC_PARAMS,
    scratch_types=[
        pltpu.VMEM((NCH, CH), jnp.int32),
        pltpu.VMEM((NCH, CH), jnp.int32),
        pltpu.VMEM((2, CH, H), jnp.float32),
        pltpu.VMEM((NPT, H), jnp.float32),
        pltpu.VMEM((NPT,), jnp.float32),
        pltpu.VMEM((NPT,), jnp.float32),
        pltpu.VMEM_SHARED((NP, H), jnp.float32),
        pltpu.SemaphoreType.DMA,
        pltpu.SemaphoreType.DMA,
    ],
)
def _agg1_kernel(src_hbm, dst_hbm, h1_hbm, degp_hbm, zeros_hbm,
                 acc_hbm, tab_hbm,
                 src_v, dst_v, rows_v, buf_v, d0_v, d1_v, acc_sh, sem0, sem1):
    c = lax.axis_index("c")
    s = lax.axis_index("s")
    wid = s * NC + c

    pltpu.sync_copy(zeros_hbm.at[pl.ds(s * NPT, NPT)], buf_v)
    pltpu.sync_copy(buf_v, acc_sh.at[pl.ds(s * NPT, NPT)])

    _load_dinv(degp_hbm, d0_v, d1_v, s)
    pltpu.sync_copy(h1_hbm.at[pl.ds(s * NPT, NPT)], buf_v)

    def scale(i, _):
        dinv = plsc.load_gather(d0_v, [jnp.full((16,), i, jnp.int32)])
        buf_v[i] = buf_v[i] * dinv
        return 0
    lax.fori_loop(0, NPT, scale, 0)
    pltpu.sync_copy(buf_v, tab_hbm.at[c, pl.ds(s * NPT, NPT)])

    _load_edges(src_hbm, dst_hbm, src_v, dst_v, wid)
    plsc.subcore_barrier()

    _edge_pass(tab_hbm.at[c], src_v, dst_v, rows_v, acc_sh, [sem0, sem1])
    plsc.subcore_barrier()

    pltpu.sync_copy(acc_sh.at[pl.ds(s * NPT, NPT)], buf_v)
    pltpu.sync_copy(buf_v, acc_hbm.at[c, pl.ds(s * NPT, NPT)])


@functools.partial(
    pl.kernel,
    out_type=(
        jax.ShapeDtypeStruct((NC, NP, C), jnp.float32),
        jax.ShapeDtypeStruct((NC, NP, H), jnp.float32),
    ),
    mesh=_sc_mesh(),
    compiler_params=_SC_PARAMS,
    scratch_types=[
        pltpu.VMEM((NCH, CH), jnp.int32),
        pltpu.VMEM((NCH, CH), jnp.int32),
        pltpu.VMEM((2, CH, H), jnp.float32),
        pltpu.VMEM((NPT, H), jnp.float32),
        pltpu.VMEM((NPT, H), jnp.float32),
        pltpu.VMEM((NPT, H), jnp.float32),
        pltpu.VMEM((NPT,), jnp.float32),
        pltpu.VMEM((NPT,), jnp.float32),
        pltpu.VMEM((16,), jnp.float32),
        pltpu.VMEM((16,), jnp.float32),
        pltpu.VMEM((16,), jnp.float32),
        pltpu.VMEM((NPT, C), jnp.float32),
        pltpu.VMEM_SHARED((NP, H), jnp.float32),
        pltpu.SemaphoreType.DMA,
        pltpu.SemaphoreType.DMA,
    ],
)
def _agg2_kernel(src_hbm, dst_hbm, h1_hbm, degp_hbm, acc_hbm, zeros_hbm,
                 b1_hbm, w2a_hbm, w2b_hbm, b2h_hbm,
                 q_hbm, tab_hbm,
                 src_v, dst_v, rows_v, u_v, a0_v, a1_v, d0_v, d1_v,
                 b1_v, w2a_v, w2b_v, q_v, acc_sh, sem0, sem1):
    c = lax.axis_index("c")
    s = lax.axis_index("s")
    wid = s * NC + c

    pltpu.sync_copy(zeros_hbm.at[pl.ds(s * NPT, NPT)], a0_v)
    pltpu.sync_copy(a0_v, acc_sh.at[pl.ds(s * NPT, NPT)])

    _load_dinv(degp_hbm, d0_v, d1_v, s)
    pltpu.sync_copy(h1_hbm.at[pl.ds(s * NPT, NPT)], u_v)
    pltpu.sync_copy(acc_hbm.at[0, pl.ds(s * NPT, NPT)], a0_v)
    pltpu.sync_copy(acc_hbm.at[1, pl.ds(s * NPT, NPT)], a1_v)
    pltpu.sync_copy(b1_hbm, b1_v)
    pltpu.sync_copy(w2a_hbm, w2a_v)
    pltpu.sync_copy(w2b_hbm, w2b_v)

    b1_vec = b1_v[...]

    def mk_u(i, _):
        dinv = plsc.load_gather(d0_v, [jnp.full((16,), i, jnp.int32)])
        h1s = u_v[i] * dinv
        out1 = (a0_v[i] + a1_v[i] + h1s) * dinv + b1_vec
        u_v[i] = jnp.maximum(out1, 0.0) * dinv
        return 0
    lax.fori_loop(0, NPT, mk_u, 0)
    pltpu.sync_copy(u_v, tab_hbm.at[c, pl.ds(s * NPT, NPT)])

    _load_edges(src_hbm, dst_hbm, src_v, dst_v, wid)
    plsc.subcore_barrier()

    _edge_pass(tab_hbm.at[c], src_v, dst_v, rows_v, acc_sh, [sem0, sem1])
    plsc.subcore_barrier()

    pltpu.sync_copy(acc_sh.at[pl.ds(s * NPT, NPT)], a0_v)
    pltpu.sync_copy(b2h_hbm, b1_v)
    flag = jnp.where(c == 0, 1.0, 0.0)
    w2a_vec = w2a_v[...]
    w2b_vec = w2b_v[...]
    b2_vec = b1_v[...]
    iota16 = lax.iota(jnp.int32, 16)

    def mk_q(g, _):
        ridx = iota16 + g * 16
        dinvg = d0_v[pl.ds(g * 16, 16)]
        qa = jnp.zeros((16,), jnp.float32)
        qb = jnp.zeros((16,), jnp.float32)
        for k in range(H):
            kidx = jnp.full((16,), k, jnp.int32)
            col = (plsc.load_gather(a0_v, [ridx, kidx])
                   + plsc.load_gather(u_v, [ridx, kidx]) * flag)
            qa = qa + col * w2a_vec[k]
            qb = qb + col * w2b_vec[k]
        qa = qa * dinvg + b2_vec[0]
        qb = qb * dinvg + b2_vec[1]
        plsc.store_scatter(q_v, [ridx, jnp.full((16,), 0, jnp.int32)], qa)
        plsc.store_scatter(q_v, [ridx, jnp.full((16,), 1, jnp.int32)], qb)
        return 0
    lax.fori_loop(0, NPT // 16, mk_q, 0)
    pltpu.sync_copy(q_v, q_hbm.at[c, pl.ds(s * NPT, NPT)])


_BLK = 512
_GRID = NP // _BLK


def _mm1_body(x_ref, w1_ref, out_ref):
    out_ref[...] = jnp.dot(x_ref[...], w1_ref[...],
                           preferred_element_type=jnp.float32)


def _mm1(x, W1):
    return pl.pallas_call(
        _mm1_body,
        grid=(_GRID,),
        in_specs=[
            pl.BlockSpec((_BLK, F_IN), lambda i: (i, 0)),
            pl.BlockSpec((F_IN, H), lambda i: (0, 0)),
        ],
        out_specs=pl.BlockSpec((_BLK, H), lambda i: (i, 0)),
        out_shape=jax.ShapeDtypeStruct((NP, H), jnp.float32),
    )(x, W1)


_FBLK = 400
_FGRID = N_NODES // _FBLK


def _final_body(q_ref, out_ref):
    o = q_ref[0] + q_ref[1]
    m = jnp.max(o, axis=1, keepdims=True)
    z = o - m
    lse = jnp.log(jnp.exp(z[:, 0:1]) + jnp.exp(z[:, 1:2]))
    out_ref[...] = z - lse


def _final(q):
    return pl.pallas_call(
        _final_body,
        grid=(_FGRID,),
        in_specs=[pl.BlockSpec((NC, _FBLK, C), lambda i: (0, i, 0))],
        out_specs=pl.BlockSpec((_FBLK, C), lambda i: (i, 0)),
        out_shape=jax.ShapeDtypeStruct((N_NODES, C), jnp.float32),
    )(q)


def kernel(x, edge_index, W1, b1, W2, b2):
    ei = edge_index.astype(jnp.int32)
    pad = jnp.full((EP - N_EDGES,), N_NODES, jnp.int32)
    src_p = jnp.concatenate([ei[0], pad])
    dst_p = jnp.concatenate([ei[1], pad])
    z16 = jnp.zeros((NP, H), jnp.float32)
    b1t = b1.astype(jnp.float32)
    w2a = W2[:, 0].astype(jnp.float32)
    w2b = W2[:, 1].astype(jnp.float32)
    b2h = jnp.tile(b2.astype(jnp.float32) * 0.5, 8)

    degp = _deg_kernel(dst_p)
    h1 = _mm1(x, W1)
    acc, h1s_tab = _agg1_kernel(src_p, dst_p, h1, degp, z16)
    del h1s_tab
    q, u_tab = _agg2_kernel(src_p, dst_p, h1, degp, acc, z16,
                            b1t, w2a, w2b, b2h)
    del u_tab
    return _final(q)


# speedup vs baseline: 1.8421x; 1.8421x over previous
import functools

import jax
import jax.numpy as jnp
from jax import lax
from jax.experimental import pallas as pl
from jax.experimental.pallas import tpu as pltpu
from jax.experimental.pallas import tpu_sc as plsc

N_NODES = 10000
N_EDGES = 320000
F_IN = 128
H = 16
C = 2

NP = 10240
NC = 2
NS = 16
EPT = N_EDGES // (NC * NS)
CH = 2000
NCH = EPT // CH
NPT = NP // NS
NPW = NP // (NC * NS)


def _sc_mesh():
    return plsc.VectorSubcoreMesh(core_axis_name="c", subcore_axis_name="s",
                                  num_cores=NC, num_subcores=NS)


_SC_PARAMS = pltpu.CompilerParams(use_tc_tiling_on_sc=False,
                                  needs_layout_passes=False)


def _rsqrt16(x):
    i = plsc.bitcast(x, jnp.int32)
    i = jnp.int32(0x5F3759DF) - lax.shift_right_logical(i, 1)
    y = plsc.bitcast(i, jnp.float32)
    hx = x * 0.5
    for _ in range(3):
        y = y * (1.5 - hx * y * y)
    return y


def _load_dinv(degp_hbm, d0_v, d1_v, s):
    pltpu.sync_copy(degp_hbm.at[0, pl.ds(s * NPT, NPT)], d0_v)
    pltpu.sync_copy(degp_hbm.at[1, pl.ds(s * NPT, NPT)], d1_v)

    def body(i, _):
        d = d0_v[pl.ds(i * 16, 16)] + d1_v[pl.ds(i * 16, 16)] + 1.0
        d0_v[pl.ds(i * 16, 16)] = _rsqrt16(d)
        return 0
    lax.fori_loop(0, NPT // 16, body, 0)


def _load_edges(src_hbm, dst_hbm, src_v, dst_v, wid):
    for j in range(NCH):
        pltpu.sync_copy(src_hbm.at[pl.ds(wid * EPT + j * CH, CH)], src_v.at[j])
        pltpu.sync_copy(dst_hbm.at[pl.ds(wid * EPT + j * CH, CH)], dst_v.at[j])


def _edge_pass(tab_ref, src_v, dst_v, rows_v, acc_sh, sems):
    def gather(j):
        return pltpu.async_copy(tab_ref.at[src_v.at[j]], rows_v.at[j % 2],
                                sems[j % 2])

    desc = {0: gather(0)}
    for j in range(NCH):
        if j + 1 < NCH:
            desc[j + 1] = gather(j + 1)
        desc[j].wait()
        pltpu.sync_copy(rows_v.at[j % 2], acc_sh.at[dst_v.at[j]], add=True)


@functools.partial(
    pl.kernel,
    out_type=jax.ShapeDtypeStruct((NC, NP), jnp.float32),
    mesh=_sc_mesh(),
    compiler_params=_SC_PARAMS,
    scratch_types=[
        pltpu.VMEM((NCH, CH), jnp.int32),
        pltpu.VMEM((CH,), jnp.float32),
        pltpu.VMEM((NPT,), jnp.float32),
        pltpu.VMEM_SHARED((NP,), jnp.float32),
    ],
)
def _deg_kernel(dst_hbm, out_hbm, idx_v, ones_v, buf_v, deg_sh):
    c = lax.axis_index("c")
    s = lax.axis_index("s")
    wid = s * NC + c

    def fill(i, _):
        buf_v[pl.ds(i * 16, 16)] = jnp.zeros((16,), jnp.float32)
        return 0
    lax.fori_loop(0, NPT // 16, fill, 0)

    def fill1(i, _):
        ones_v[pl.ds(i * 16, 16)] = jnp.ones((16,), jnp.float32)
        return 0
    lax.fori_loop(0, CH // 16, fill1, 0)

    pltpu.sync_copy(buf_v, deg_sh.at[pl.ds(s * NPT, NPT)])
    for j in range(NCH):
        pltpu.sync_copy(dst_hbm.at[pl.ds(wid * EPT + j * CH, CH)], idx_v.at[j])
    plsc.subcore_barrier()

    for j in range(NCH):
        pltpu.sync_copy(ones_v, deg_sh.at[idx_v.at[j]], add=True)
    plsc.subcore_barrier()

    pltpu.sync_copy(deg_sh.at[pl.ds(s * NPT, NPT)], buf_v)
    pltpu.sync_copy(buf_v, out_hbm.at[c, pl.ds(s * NPT, NPT)])


@functools.partial(
    pl.kernel,
    out_type=(
        jax.ShapeDtypeStruct((NC, NP, H), jnp.float32),
        jax.ShapeDtypeStruct((NC, NP, H), jnp.float32),
    ),
    mesh=_sc_mesh(),
    compiler_params=_SC_PARAMS,
    scratch_types=[
        pltpu.VMEM((NCH, CH), jnp.int32),
        pltpu.VMEM((NCH, CH), jnp.int32),
        pltpu.VMEM((2, CH, H), jnp.float32),
        pltpu.VMEM((NPT, H), jnp.float32),
        pltpu.VMEM((NPT,), jnp.float32),
        pltpu.VMEM((NPT,), jnp.float32),
        pltpu.VMEM_SHARED((NP, H), jnp.float32),
        pltpu.SemaphoreType.DMA,
        pltpu.SemaphoreType.DMA,
    ],
)
def _agg1_kernel(src_hbm, dst_hbm, h1_hbm, degp_hbm, zeros_hbm,
                 acc_hbm, tab_hbm,
                 src_v, dst_v, rows_v, buf_v, d0_v, d1_v, acc_sh, sem0, sem1):
    c = lax.axis_index("c")
    s = lax.axis_index("s")
    wid = s * NC + c

    pltpu.sync_copy(zeros_hbm.at[pl.ds(s * NPT, NPT)], buf_v)
    pltpu.sync_copy(buf_v, acc_sh.at[pl.ds(s * NPT, NPT)])

    _load_dinv(degp_hbm, d0_v, d1_v, s)
    pltpu.sync_copy(h1_hbm.at[pl.ds(s * NPT, NPT)], buf_v)

    def scale(i, _):
        dinv = plsc.load_gather(d0_v, [jnp.full((16,), i, jnp.int32)])
        buf_v[i] = buf_v[i] * dinv
        return 0
    lax.fori_loop(0, NPT, scale, 0)
    pltpu.sync_copy(buf_v, tab_hbm.at[c, pl.ds(s * NPT, NPT)])

    _load_edges(src_hbm, dst_hbm, src_v, dst_v, wid)
    plsc.subcore_barrier()

    _edge_pass(tab_hbm.at[c], src_v, dst_v, rows_v, acc_sh, [sem0, sem1])
    plsc.subcore_barrier()

    pltpu.sync_copy(acc_sh.at[pl.ds(s * NPT, NPT)], buf_v)
    pltpu.sync_copy(buf_v, acc_hbm.at[c, pl.ds(s * NPT, NPT)])


@functools.partial(
    pl.kernel,
    out_type=(
        jax.ShapeDtypeStruct((NC, C, NP), jnp.float32),
        jax.ShapeDtypeStruct((NC, NP, H), jnp.float32),
    ),
    mesh=_sc_mesh(),
    compiler_params=_SC_PARAMS,
    scratch_types=[
        pltpu.VMEM((NCH, CH), jnp.int32),
        pltpu.VMEM((NCH, CH), jnp.int32),
        pltpu.VMEM((2, CH, H), jnp.float32),
        pltpu.VMEM((NPT, H), jnp.float32),
        pltpu.VMEM((NPT, H), jnp.float32),
        pltpu.VMEM((NPT, H), jnp.float32),
        pltpu.VMEM((NPT,), jnp.float32),
        pltpu.VMEM((NPT,), jnp.float32),
        pltpu.VMEM((16,), jnp.float32),
        pltpu.VMEM((16,), jnp.float32),
        pltpu.VMEM((16,), jnp.float32),
        pltpu.VMEM((NPT,), jnp.float32),
        pltpu.VMEM((NPT,), jnp.float32),
        pltpu.VMEM_SHARED((NP, H), jnp.float32),
        pltpu.SemaphoreType.DMA,
        pltpu.SemaphoreType.DMA,
    ],
)
def _agg2_kernel(src_hbm, dst_hbm, h1_hbm, degp_hbm, acc_hbm, zeros_hbm,
                 b1_hbm, w2a_hbm, w2b_hbm, b2h_hbm,
                 q_hbm, tab_hbm,
                 src_v, dst_v, rows_v, u_v, a0_v, a1_v, d0_v, d1_v,
                 b1_v, w2a_v, w2b_v, qa_v, qb_v, acc_sh, sem0, sem1):
    c = lax.axis_index("c")
    s = lax.axis_index("s")
    wid = s * NC + c

    pltpu.sync_copy(zeros_hbm.at[pl.ds(s * NPT, NPT)], a0_v)
    pltpu.sync_copy(a0_v, acc_sh.at[pl.ds(s * NPT, NPT)])

    _load_dinv(degp_hbm, d0_v, d1_v, s)
    pltpu.sync_copy(h1_hbm.at[pl.ds(s * NPT, NPT)], u_v)
    pltpu.sync_copy(acc_hbm.at[0, pl.ds(s * NPT, NPT)], a0_v)
    pltpu.sync_copy(acc_hbm.at[1, pl.ds(s * NPT, NPT)], a1_v)
    pltpu.sync_copy(b1_hbm, b1_v)
    pltpu.sync_copy(w2a_hbm, w2a_v)
    pltpu.sync_copy(w2b_hbm, w2b_v)
    b1_vec = b1_v[...]

    def mk_u(i, _):
        dinv = plsc.load_gather(d0_v, [jnp.full((16,), i, jnp.int32)])
        h1s = u_v[i] * dinv
        out1 = (a0_v[i] + a1_v[i] + h1s) * dinv + b1_vec
        u_v[i] = jnp.maximum(out1, 0.0) * dinv
        return 0
    lax.fori_loop(0, NPT, mk_u, 0)
    pltpu.sync_copy(u_v, tab_hbm.at[c, pl.ds(s * NPT, NPT)])

    _load_edges(src_hbm, dst_hbm, src_v, dst_v, wid)
    plsc.subcore_barrier()

    _edge_pass(tab_hbm.at[c], src_v, dst_v, rows_v, acc_sh, [sem0, sem1])
    plsc.subcore_barrier()

    pltpu.sync_copy(acc_sh.at[pl.ds(s * NPT, NPT)], a0_v)
    pltpu.sync_copy(b2h_hbm, b1_v)
    flag = jnp.where(c == 0, 1.0, 0.0)
    w2a_vec = w2a_v[...]
    w2b_vec = w2b_v[...]
    b2_vec = b1_v[...]
    iota16 = lax.iota(jnp.int32, 16)

    def mk_q(g, _):
        ridx = iota16 + g * 16
        dinvg = d0_v[pl.ds(g * 16, 16)]
        qa = jnp.zeros((16,), jnp.float32)
        qb = jnp.zeros((16,), jnp.float32)
        for k in range(H):
            kidx = jnp.full((16,), k, jnp.int32)
            col = (plsc.load_gather(a0_v, [ridx, kidx])
                   + plsc.load_gather(u_v, [ridx, kidx]) * flag)
            qa = qa + col * w2a_vec[k]
            qb = qb + col * w2b_vec[k]
        qa_v[pl.ds(g * 16, 16)] = qa * dinvg + b2_vec[0]
        qb_v[pl.ds(g * 16, 16)] = qb * dinvg + b2_vec[1]
        return 0
    lax.fori_loop(0, NPT // 16, mk_q, 0)
    pltpu.sync_copy(qa_v, q_hbm.at[c, 0, pl.ds(s * NPT, NPT)])
    pltpu.sync_copy(qb_v, q_hbm.at[c, 1, pl.ds(s * NPT, NPT)])


@functools.partial(
    pl.kernel,
    out_type=jax.ShapeDtypeStruct((NP * C,), jnp.float32),
    mesh=_sc_mesh(),
    compiler_params=_SC_PARAMS,
    scratch_types=[
        pltpu.VMEM((NPW,), jnp.float32),
        pltpu.VMEM((NPW,), jnp.float32),
        pltpu.VMEM((NPW,), jnp.float32),
        pltpu.VMEM((NPW * C,), jnp.float32),
    ],
)
def _final_kernel(q_hbm, out_hbm, qa_v, qb_v, t_v, out_v):
    c = lax.axis_index("c")
    s = lax.axis_index("s")
    wid = s * NC + c
    base = wid * NPW

    pltpu.sync_copy(q_hbm.at[0, 0, pl.ds(base, NPW)], qa_v)
    pltpu.sync_copy(q_hbm.at[1, 0, pl.ds(base, NPW)], t_v)

    def add_a(i, _):
        sl = pl.ds(i * 16, 16)
        qa_v[sl] = qa_v[sl] + t_v[sl]
        return 0
    lax.fori_loop(0, NPW // 16, add_a, 0)

    pltpu.sync_copy(q_hbm.at[0, 1, pl.ds(base, NPW)], qb_v)
    pltpu.sync_copy(q_hbm.at[1, 1, pl.ds(base, NPW)], t_v)

    def add_b(i, _):
        sl = pl.ds(i * 16, 16)
        qb_v[sl] = qb_v[sl] + t_v[sl]
        return 0
    lax.fori_loop(0, NPW // 16, add_b, 0)

    iota16 = lax.iota(jnp.int32, 16)

    def lsm(g, _):
        sl = pl.ds(g * 16, 16)
        a = qa_v[sl]
        b = qb_v[sl]
        m = jnp.maximum(a, b)
        e = jnp.exp(jnp.minimum(a, b) - m)
        t = e / (2.0 + e)
        t2 = t * t
        ln = 2.0 * t * (1.0 + t2 * (1.0 / 3.0 + t2 * (0.2 + t2 * (
            1.0 / 7.0 + t2 * (1.0 / 9.0)))))
        oidx = iota16 * 2 + g * 32
        plsc.store_scatter(out_v, [oidx], a - m - ln)
        plsc.store_scatter(out_v, [oidx + 1], b - m - ln)
        return 0
    lax.fori_loop(0, NPW // 16, lsm, 0)
    pltpu.sync_copy(out_v, out_hbm.at[pl.ds(base * C, NPW * C)])


_BLK = 512
_GRID = NP // _BLK


def _mm1_body(x_ref, w1_ref, out_ref):
    out_ref[...] = jnp.dot(x_ref[...], w1_ref[...],
                           preferred_element_type=jnp.float32)


def _mm1(x, W1):
    return pl.pallas_call(
        _mm1_body,
        grid=(_GRID,),
        in_specs=[
            pl.BlockSpec((_BLK, F_IN), lambda i: (i, 0)),
            pl.BlockSpec((F_IN, H), lambda i: (0, 0)),
        ],
        out_specs=pl.BlockSpec((_BLK, H), lambda i: (i, 0)),
        out_shape=jax.ShapeDtypeStruct((NP, H), jnp.float32),
    )(x, W1)


def kernel(x, edge_index, W1, b1, W2, b2):
    ei = edge_index.astype(jnp.int32)
    src = ei[0]
    dst = ei[1]
    z16 = jnp.zeros((NP, H), jnp.float32)
    b1t = b1.astype(jnp.float32)
    w2a = W2[:, 0].astype(jnp.float32)
    w2b = W2[:, 1].astype(jnp.float32)
    b2h = jnp.tile(b2.astype(jnp.float32) * 0.5, 8)

    degp = _deg_kernel(dst)
    h1 = _mm1(x, W1)
    acc, h1s_tab = _agg1_kernel(src, dst, h1, degp, z16)
    del h1s_tab
    q, u_tab = _agg2_kernel(src, dst, h1, degp, acc, z16, b1t, w2a, w2b, b2h)
    del u_tab
    out = _final_kernel(q)
    return out[:N_NODES * C].reshape(N_NODES, C)


# speedup vs baseline: 1.9781x; 1.0738x over previous
import functools

import jax
import jax.numpy as jnp
from jax import lax
from jax.experimental import pallas as pl
from jax.experimental.pallas import tpu as pltpu
from jax.experimental.pallas import tpu_sc as plsc

N_NODES = 10000
N_EDGES = 320000
F_IN = 128
H = 16
C = 2

NP = 10240
NC = 2
NS = 16
EPT = N_EDGES // (NC * NS)
CH = 2000
NCH = EPT // CH
NPT = NP // NS
NPW = NP // (NC * NS)


def _sc_mesh():
    return plsc.VectorSubcoreMesh(core_axis_name="c", subcore_axis_name="s",
                                  num_cores=NC, num_subcores=NS)


_SC_PARAMS = pltpu.CompilerParams(use_tc_tiling_on_sc=False,
                                  needs_layout_passes=False)


def _rsqrt16(x):
    i = plsc.bitcast(x, jnp.int32)
    i = jnp.int32(0x5F3759DF) - lax.shift_right_logical(i, 1)
    y = plsc.bitcast(i, jnp.float32)
    hx = x * 0.5
    for _ in range(3):
        y = y * (1.5 - hx * y * y)
    return y


def _load_dinv(degp_hbm, d0_v, d1_v, s):
    pltpu.sync_copy(degp_hbm.at[0, pl.ds(s * NPT, NPT)], d0_v)
    pltpu.sync_copy(degp_hbm.at[1, pl.ds(s * NPT, NPT)], d1_v)

    def body(i, _):
        d = d0_v[pl.ds(i * 16, 16)] + d1_v[pl.ds(i * 16, 16)] + 1.0
        d0_v[pl.ds(i * 16, 16)] = _rsqrt16(d)
        return 0
    lax.fori_loop(0, NPT // 16, body, 0)


def _load_edges(src_hbm, dst_hbm, src_v, dst_v, wid):
    for j in range(NCH):
        pltpu.sync_copy(src_hbm.at[pl.ds(wid * EPT + j * CH, CH)], src_v.at[j])
        pltpu.sync_copy(dst_hbm.at[pl.ds(wid * EPT + j * CH, CH)], dst_v.at[j])


def _edge_pass(tab_ref, src_v, dst_v, rows_v, acc_sh, sems):
    def gather(j):
        return pltpu.async_copy(tab_ref.at[src_v.at[j]], rows_v.at[j % 2],
                                sems[j % 2])

    desc = {0: gather(0)}
    for j in range(NCH):
        if j + 1 < NCH:
            desc[j + 1] = gather(j + 1)
        desc[j].wait()
        pltpu.sync_copy(rows_v.at[j % 2], acc_sh.at[dst_v.at[j]], add=True)


@functools.partial(
    pl.kernel,
    out_type=jax.ShapeDtypeStruct((NC, NP), jnp.float32),
    mesh=_sc_mesh(),
    compiler_params=_SC_PARAMS,
    scratch_types=[
        pltpu.VMEM((NCH, CH), jnp.int32),
        pltpu.VMEM((CH,), jnp.float32),
        pltpu.VMEM((NPT,), jnp.float32),
        pltpu.VMEM_SHARED((NP,), jnp.float32),
    ],
)
def _deg_kernel(dst_hbm, out_hbm, idx_v, ones_v, buf_v, deg_sh):
    c = lax.axis_index("c")
    s = lax.axis_index("s")
    wid = s * NC + c

    def fill(i, _):
        buf_v[pl.ds(i * 16, 16)] = jnp.zeros((16,), jnp.float32)
        return 0
    lax.fori_loop(0, NPT // 16, fill, 0)

    def fill1(i, _):
        ones_v[pl.ds(i * 16, 16)] = jnp.ones((16,), jnp.float32)
        return 0
    lax.fori_loop(0, CH // 16, fill1, 0)

    pltpu.sync_copy(buf_v, deg_sh.at[pl.ds(s * NPT, NPT)])
    for j in range(NCH):
        pltpu.sync_copy(dst_hbm.at[pl.ds(wid * EPT + j * CH, CH)], idx_v.at[j])
    plsc.subcore_barrier()

    for j in range(NCH):
        pltpu.sync_copy(ones_v, deg_sh.at[idx_v.at[j]], add=True)
    plsc.subcore_barrier()

    pltpu.sync_copy(deg_sh.at[pl.ds(s * NPT, NPT)], buf_v)
    pltpu.sync_copy(buf_v, out_hbm.at[c, pl.ds(s * NPT, NPT)])


@functools.partial(
    pl.kernel,
    out_type=(
        jax.ShapeDtypeStruct((NC, NP, H), jnp.float32),
        jax.ShapeDtypeStruct((NC, NP, H), jnp.float32),
    ),
    mesh=_sc_mesh(),
    compiler_params=_SC_PARAMS,
    scratch_types=[
        pltpu.VMEM((NCH, CH), jnp.int32),
        pltpu.VMEM((NCH, CH), jnp.int32),
        pltpu.VMEM((2, CH, H), jnp.float32),
        pltpu.VMEM((NPT, H), jnp.float32),
        pltpu.VMEM((NPT,), jnp.float32),
        pltpu.VMEM((NPT,), jnp.float32),
        pltpu.VMEM_SHARED((NP, H), jnp.float32),
        pltpu.SemaphoreType.DMA,
        pltpu.SemaphoreType.DMA,
    ],
)
def _agg1_kernel(src_hbm, dst_hbm, h1_hbm, degp_hbm, zeros_hbm,
                 acc_hbm, tab_hbm,
                 src_v, dst_v, rows_v, buf_v, d0_v, d1_v, acc_sh, sem0, sem1):
    c = lax.axis_index("c")
    s = lax.axis_index("s")
    wid = s * NC + c

    pltpu.sync_copy(zeros_hbm.at[pl.ds(s * NPT, NPT)], buf_v)
    pltpu.sync_copy(buf_v, acc_sh.at[pl.ds(s * NPT, NPT)])

    _load_dinv(degp_hbm, d0_v, d1_v, s)
    pltpu.sync_copy(h1_hbm.at[pl.ds(s * NPT, NPT)], buf_v)

    def scale(i, _):
        dinv = plsc.load_gather(d0_v, [jnp.full((16,), i, jnp.int32)])
        buf_v[i] = buf_v[i] * dinv
        return 0
    lax.fori_loop(0, NPT, scale, 0)
    pltpu.sync_copy(buf_v, tab_hbm.at[c, pl.ds(s * NPT, NPT)])

    _load_edges(src_hbm, dst_hbm, src_v, dst_v, wid)
    plsc.subcore_barrier()

    _edge_pass(tab_hbm.at[c], src_v, dst_v, rows_v, acc_sh, [sem0, sem1])
    plsc.subcore_barrier()

    pltpu.sync_copy(acc_sh.at[pl.ds(s * NPT, NPT)], buf_v)
    pltpu.sync_copy(buf_v, acc_hbm.at[c, pl.ds(s * NPT, NPT)])


@functools.partial(
    pl.kernel,
    out_type=(
        jax.ShapeDtypeStruct((NC, C, NP), jnp.float32),
        jax.ShapeDtypeStruct((NC, NP, H), jnp.float32),
    ),
    mesh=_sc_mesh(),
    compiler_params=_SC_PARAMS,
    scratch_types=[
        pltpu.VMEM((NCH, CH), jnp.int32),
        pltpu.VMEM((NCH, CH), jnp.int32),
        pltpu.VMEM((2, CH, H), jnp.float32),
        pltpu.VMEM((NPT, H), jnp.float32),
        pltpu.VMEM((NPT, H), jnp.float32),
        pltpu.VMEM((NPT, H), jnp.float32),
        pltpu.VMEM((NPT,), jnp.float32),
        pltpu.VMEM((NPT,), jnp.float32),
        pltpu.VMEM((16,), jnp.float32),
        pltpu.VMEM((16,), jnp.float32),
        pltpu.VMEM((16,), jnp.float32),
        pltpu.VMEM((NPT,), jnp.float32),
        pltpu.VMEM((NPT,), jnp.float32),
        pltpu.VMEM_SHARED((NP, H), jnp.float32),
        pltpu.SemaphoreType.DMA,
        pltpu.SemaphoreType.DMA,
    ],
)
def _agg2_kernel(src_hbm, dst_hbm, h1_hbm, degp_hbm, acc_hbm, zeros_hbm,
                 b1_hbm, w2a_hbm, w2b_hbm, b2h_hbm,
                 q_hbm, tab_hbm,
                 src_v, dst_v, rows_v, u_v, a0_v, a1_v, d0_v, d1_v,
                 b1_v, w2a_v, w2b_v, qa_v, qb_v, acc_sh, sem0, sem1):
    c = lax.axis_index("c")
    s = lax.axis_index("s")
    wid = s * NC + c

    pltpu.sync_copy(zeros_hbm.at[pl.ds(s * NPT, NPT)], a0_v)
    pltpu.sync_copy(a0_v, acc_sh.at[pl.ds(s * NPT, NPT)])

    _load_dinv(degp_hbm, d0_v, d1_v, s)
    pltpu.sync_copy(h1_hbm.at[pl.ds(s * NPT, NPT)], u_v)
    pltpu.sync_copy(acc_hbm.at[0, pl.ds(s * NPT, NPT)], a0_v)
    pltpu.sync_copy(acc_hbm.at[1, pl.ds(s * NPT, NPT)], a1_v)
    pltpu.sync_copy(b1_hbm, b1_v)
    pltpu.sync_copy(w2a_hbm, w2a_v)
    pltpu.sync_copy(w2b_hbm, w2b_v)
    b1_vec = b1_v[...]

    def mk_u(i, _):
        dinv = plsc.load_gather(d0_v, [jnp.full((16,), i, jnp.int32)])
        h1s = u_v[i] * dinv
        out1 = (a0_v[i] + a1_v[i] + h1s) * dinv + b1_vec
        u_v[i] = jnp.maximum(out1, 0.0) * dinv
        return 0
    lax.fori_loop(0, NPT, mk_u, 0)
    pltpu.sync_copy(u_v, tab_hbm.at[c, pl.ds(s * NPT, NPT)])

    _load_edges(src_hbm, dst_hbm, src_v, dst_v, wid)
    plsc.subcore_barrier()

    _edge_pass(tab_hbm.at[c], src_v, dst_v, rows_v, acc_sh, [sem0, sem1])
    plsc.subcore_barrier()

    pltpu.sync_copy(acc_sh.at[pl.ds(s * NPT, NPT)], a0_v)
    pltpu.sync_copy(b2h_hbm, b1_v)
    flag = jnp.where(c == 0, 1.0, 0.0)
    w2a_vec = w2a_v[...]
    w2b_vec = w2b_v[...]
    b2_vec = b1_v[...]
    iota16 = lax.iota(jnp.int32, 16)

    def mk_q(g, _):
        ridx = iota16 + g * 16
        dinvg = d0_v[pl.ds(g * 16, 16)]
        qa = jnp.zeros((16,), jnp.float32)
        qb = jnp.zeros((16,), jnp.float32)
        for k in range(H):
            kidx = jnp.full((16,), k, jnp.int32)
            col = (plsc.load_gather(a0_v, [ridx, kidx])
                   + plsc.load_gather(u_v, [ridx, kidx]) * flag)
            qa = qa + col * w2a_vec[k]
            qb = qb + col * w2b_vec[k]
        qa_v[pl.ds(g * 16, 16)] = qa * dinvg + b2_vec[0]
        qb_v[pl.ds(g * 16, 16)] = qb * dinvg + b2_vec[1]
        return 0
    lax.fori_loop(0, NPT // 16, mk_q, 0)
    pltpu.sync_copy(qa_v, q_hbm.at[c, 0, pl.ds(s * NPT, NPT)])
    pltpu.sync_copy(qb_v, q_hbm.at[c, 1, pl.ds(s * NPT, NPT)])


@functools.partial(
    pl.kernel,
    out_type=jax.ShapeDtypeStruct((NP * C,), jnp.float32),
    mesh=_sc_mesh(),
    compiler_params=_SC_PARAMS,
    scratch_types=[
        pltpu.VMEM((NPW,), jnp.float32),
        pltpu.VMEM((NPW,), jnp.float32),
        pltpu.VMEM((NPW,), jnp.float32),
        pltpu.VMEM((NPW * C,), jnp.float32),
    ],
)
def _final_kernel(q_hbm, out_hbm, qa_v, qb_v, t_v, out_v):
    c = lax.axis_index("c")
    s = lax.axis_index("s")
    wid = s * NC + c
    base = wid * NPW

    pltpu.sync_copy(q_hbm.at[0, 0, pl.ds(base, NPW)], qa_v)
    pltpu.sync_copy(q_hbm.at[1, 0, pl.ds(base, NPW)], t_v)

    def add_a(i, _):
        sl = pl.ds(i * 16, 16)
        qa_v[sl] = qa_v[sl] + t_v[sl]
        return 0
    lax.fori_loop(0, NPW // 16, add_a, 0)

    pltpu.sync_copy(q_hbm.at[0, 1, pl.ds(base, NPW)], qb_v)
    pltpu.sync_copy(q_hbm.at[1, 1, pl.ds(base, NPW)], t_v)

    def add_b(i, _):
        sl = pl.ds(i * 16, 16)
        qb_v[sl] = qb_v[sl] + t_v[sl]
        return 0
    lax.fori_loop(0, NPW // 16, add_b, 0)

    iota16 = lax.iota(jnp.int32, 16)

    def lsm(g, _):
        sl = pl.ds(g * 16, 16)
        a = qa_v[sl]
        b = qb_v[sl]
        m = jnp.maximum(a, b)
        e = jnp.exp(jnp.minimum(a, b) - m)
        t = e / (2.0 + e)
        t2 = t * t
        ln = 2.0 * t * (1.0 + t2 * (1.0 / 3.0 + t2 * (0.2 + t2 * (
            1.0 / 7.0 + t2 * (1.0 / 9.0)))))
        oidx = iota16 * 2 + g * 32
        plsc.store_scatter(out_v, [oidx], a - m - ln)
        plsc.store_scatter(out_v, [oidx + 1], b - m - ln)
        return 0
    lax.fori_loop(0, NPW // 16, lsm, 0)
    pltpu.sync_copy(out_v, out_hbm.at[pl.ds(base * C, NPW * C)])


_EBLK = 65536


def _esplit_body(ei_ref, src_ref, dst_ref):
    src_ref[...] = ei_ref[0]
    dst_ref[...] = ei_ref[1]


def _edge_split(ei):
    return pl.pallas_call(
        _esplit_body,
        grid=(pl.cdiv(N_EDGES, _EBLK),),
        in_specs=[pl.BlockSpec((2, _EBLK), lambda i: (0, i))],
        out_specs=(
            pl.BlockSpec((_EBLK,), lambda i: (i,)),
            pl.BlockSpec((_EBLK,), lambda i: (i,)),
        ),
        out_shape=(
            jax.ShapeDtypeStruct((N_EDGES,), jnp.int32),
            jax.ShapeDtypeStruct((N_EDGES,), jnp.int32),
        ),
    )(ei)


_BLK = 512
_GRID = NP // _BLK


def _mm1_body(x_ref, w1_ref, out_ref):
    out_ref[...] = jnp.dot(x_ref[...], w1_ref[...],
                           preferred_element_type=jnp.float32)


def _mm1(x, W1):
    return pl.pallas_call(
        _mm1_body,
        grid=(_GRID,),
        in_specs=[
            pl.BlockSpec((_BLK, F_IN), lambda i: (i, 0)),
            pl.BlockSpec((F_IN, H), lambda i: (0, 0)),
        ],
        out_specs=pl.BlockSpec((_BLK, H), lambda i: (i, 0)),
        out_shape=jax.ShapeDtypeStruct((NP, H), jnp.float32),
    )(x, W1)


def kernel(x, edge_index, W1, b1, W2, b2):
    ei = edge_index.astype(jnp.int32)
    src, dst = _edge_split(ei)
    z16 = jnp.zeros((NP, H), jnp.float32)
    b1t = b1.astype(jnp.float32)
    w2a = W2[:, 0].astype(jnp.float32)
    w2b = W2[:, 1].astype(jnp.float32)
    b2h = jnp.tile(b2.astype(jnp.float32) * 0.5, 8)

    degp = _deg_kernel(dst)
    h1 = _mm1(x, W1)
    acc, h1s_tab = _agg1_kernel(src, dst, h1, degp, z16)
    del h1s_tab
    q, u_tab = _agg2_kernel(src, dst, h1, degp, acc, z16, b1t, w2a, w2b, b2h)
    del u_tab
    out = _final_kernel(q)
    return out[:N_NODES * C].reshape(N_NODES, C)


# speedup vs baseline: 2.3139x; 1.1697x over previous
import functools

import jax
import jax.numpy as jnp
from jax import lax
from jax.experimental import pallas as pl
from jax.experimental.pallas import tpu as pltpu
from jax.experimental.pallas import tpu_sc as plsc

N_NODES = 10000
N_EDGES = 320000
F_IN = 128
H = 16
C = 2

NP = 10240
NC = 2
NS = 16
EPT = N_EDGES // (NC * NS)
CH = 2000
NCH = EPT // CH
NPT = NP // NS
NPW = NP // (NC * NS)


def _sc_mesh():
    return plsc.VectorSubcoreMesh(core_axis_name="c", subcore_axis_name="s",
                                  num_cores=NC, num_subcores=NS)


_SC_PARAMS = pltpu.CompilerParams(use_tc_tiling_on_sc=False,
                                  needs_layout_passes=False)


def _rsqrt16(x):
    i = plsc.bitcast(x, jnp.int32)
    i = jnp.int32(0x5F3759DF) - lax.shift_right_logical(i, 1)
    y = plsc.bitcast(i, jnp.float32)
    hx = x * 0.5
    for _ in range(3):
        y = y * (1.5 - hx * y * y)
    return y


def _mk_dinv(d0_v, d1_v):
    def body(i, _):
        d = d0_v[pl.ds(i * 16, 16)] + d1_v[pl.ds(i * 16, 16)] + 1.0
        d0_v[pl.ds(i * 16, 16)] = _rsqrt16(d)
        return 0
    lax.fori_loop(0, NPT // 16, body, 0)


def _load_edges(src_hbm, dst_hbm, src_v, dst_v, wid, sem):
    ds = []
    for j in range(NCH):
        ds.append(pltpu.async_copy(
            src_hbm.at[pl.ds(wid * EPT + j * CH, CH)], src_v.at[j], sem))
        ds.append(pltpu.async_copy(
            dst_hbm.at[pl.ds(wid * EPT + j * CH, CH)], dst_v.at[j], sem))
    return ds


def _zero_acc(rows_v, acc_sh, s, width):
    def fill(i, _):
        rows_v[0, i] = jnp.zeros((width,), jnp.float32)
        return 0
    lax.fori_loop(0, NPT, fill, 0)
    pltpu.sync_copy(rows_v.at[0, pl.ds(0, NPT)],
                    acc_sh.at[pl.ds(s * NPT, NPT)])


def _edge_pass(tab_ref, src_v, dst_v, rows_v, acc_sh, sems):
    def gather(j):
        return pltpu.async_copy(tab_ref.at[src_v.at[j]], rows_v.at[j % 2],
                                sems[j % 2])

    desc = {0: gather(0)}
    for j in range(NCH):
        if j + 1 < NCH:
            desc[j + 1] = gather(j + 1)
        desc[j].wait()
        pltpu.sync_copy(rows_v.at[j % 2], acc_sh.at[dst_v.at[j]], add=True)


@functools.partial(
    pl.kernel,
    out_type=jax.ShapeDtypeStruct((NC, NP), jnp.float32),
    mesh=_sc_mesh(),
    compiler_params=_SC_PARAMS,
    scratch_types=[
        pltpu.VMEM((NCH, CH), jnp.int32),
        pltpu.VMEM((CH,), jnp.float32),
        pltpu.VMEM((NPT,), jnp.float32),
        pltpu.VMEM_SHARED((NP,), jnp.float32),
        pltpu.SemaphoreType.DMA,
    ],
)
def _deg_kernel(dst_hbm, out_hbm, idx_v, ones_v, buf_v, deg_sh, sem):
    c = lax.axis_index("c")
    s = lax.axis_index("s")
    wid = s * NC + c

    ds = []
    for j in range(NCH):
        ds.append(pltpu.async_copy(
            dst_hbm.at[pl.ds(wid * EPT + j * CH, CH)], idx_v.at[j], sem))

    def fill(i, _):
        buf_v[pl.ds(i * 16, 16)] = jnp.zeros((16,), jnp.float32)
        return 0
    lax.fori_loop(0, NPT // 16, fill, 0)

    def fill1(i, _):
        ones_v[pl.ds(i * 16, 16)] = jnp.ones((16,), jnp.float32)
        return 0
    lax.fori_loop(0, CH // 16, fill1, 0)

    pltpu.sync_copy(buf_v, deg_sh.at[pl.ds(s * NPT, NPT)])
    for d in ds:
        d.wait()
    plsc.subcore_barrier()

    for j in range(NCH):
        pltpu.sync_copy(ones_v, deg_sh.at[idx_v.at[j]], add=True)
    plsc.subcore_barrier()

    pltpu.sync_copy(deg_sh.at[pl.ds(s * NPT, NPT)], buf_v)
    pltpu.sync_copy(buf_v, out_hbm.at[c, pl.ds(s * NPT, NPT)])


@functools.partial(
    pl.kernel,
    out_type=(
        jax.ShapeDtypeStruct((NC, NP, H), jnp.float32),
        jax.ShapeDtypeStruct((NC, NP, H), jnp.float32),
    ),
    mesh=_sc_mesh(),
    compiler_params=_SC_PARAMS,
    scratch_types=[
        pltpu.VMEM((NCH, CH), jnp.int32),
        pltpu.VMEM((NCH, CH), jnp.int32),
        pltpu.VMEM((2, CH, H), jnp.float32),
        pltpu.VMEM((NPT, H), jnp.float32),
        pltpu.VMEM((NPT,), jnp.float32),
        pltpu.VMEM((NPT,), jnp.float32),
        pltpu.VMEM_SHARED((NP, H), jnp.float32),
        pltpu.SemaphoreType.DMA,
        pltpu.SemaphoreType.DMA,
        pltpu.SemaphoreType.DMA,
    ],
)
def _agg1_kernel(src_hbm, dst_hbm, h1_hbm, degp_hbm,
                 acc_hbm, tab_hbm,
                 src_v, dst_v, rows_v, buf_v, d0_v, d1_v, acc_sh,
                 sem0, sem1, lsem):
    c = lax.axis_index("c")
    s = lax.axis_index("s")
    wid = s * NC + c

    ds = _load_edges(src_hbm, dst_hbm, src_v, dst_v, wid, lsem)
    dh = pltpu.async_copy(h1_hbm.at[pl.ds(s * NPT, NPT)], buf_v, sem0)
    dd0 = pltpu.async_copy(degp_hbm.at[0, pl.ds(s * NPT, NPT)], d0_v, sem1)
    dd1 = pltpu.async_copy(degp_hbm.at[1, pl.ds(s * NPT, NPT)], d1_v, sem1)

    _zero_acc(rows_v, acc_sh, s, H)

    dd0.wait()
    dd1.wait()
    _mk_dinv(d0_v, d1_v)
    dh.wait()

    def scale(i, _):
        dinv = plsc.load_gather(d0_v, [jnp.full((16,), i, jnp.int32)])
        buf_v[i] = buf_v[i] * dinv
        return 0
    lax.fori_loop(0, NPT, scale, 0)
    pltpu.sync_copy(buf_v, tab_hbm.at[c, pl.ds(s * NPT, NPT)])

    for d in ds:
        d.wait()
    plsc.subcore_barrier()

    _edge_pass(tab_hbm.at[c], src_v, dst_v, rows_v, acc_sh, [sem0, sem1])
    plsc.subcore_barrier()

    pltpu.sync_copy(acc_sh.at[pl.ds(s * NPT, NPT)], buf_v)
    pltpu.sync_copy(buf_v, acc_hbm.at[c, pl.ds(s * NPT, NPT)])


@functools.partial(
    pl.kernel,
    out_type=(
        jax.ShapeDtypeStruct((NC, C, NP), jnp.float32),
        jax.ShapeDtypeStruct((NC, NP, H), jnp.float32),
    ),
    mesh=_sc_mesh(),
    compiler_params=_SC_PARAMS,
    scratch_types=[
        pltpu.VMEM((NCH, CH), jnp.int32),
        pltpu.VMEM((NCH, CH), jnp.int32),
        pltpu.VMEM((2, CH, H), jnp.float32),
        pltpu.VMEM((NPT, H), jnp.float32),
        pltpu.VMEM((NPT, H), jnp.float32),
        pltpu.VMEM((NPT, H), jnp.float32),
        pltpu.VMEM((NPT,), jnp.float32),
        pltpu.VMEM((NPT,), jnp.float32),
        pltpu.VMEM((16,), jnp.float32),
        pltpu.VMEM((16,), jnp.float32),
        pltpu.VMEM((16,), jnp.float32),
        pltpu.VMEM((NPT,), jnp.float32),
        pltpu.VMEM((NPT,), jnp.float32),
        pltpu.VMEM_SHARED((NP, H), jnp.float32),
        pltpu.SemaphoreType.DMA,
        pltpu.SemaphoreType.DMA,
        pltpu.SemaphoreType.DMA,
    ],
)
def _agg2_kernel(src_hbm, dst_hbm, h1_hbm, degp_hbm, acc_hbm,
                 b1_hbm, w2a_hbm, w2b_hbm, b2h_hbm,
                 q_hbm, tab_hbm,
                 src_v, dst_v, rows_v, u_v, a0_v, a1_v, d0_v, d1_v,
                 b1_v, w2a_v, w2b_v, qa_v, qb_v, acc_sh, sem0, sem1, lsem):
    c = lax.axis_index("c")
    s = lax.axis_index("s")
    wid = s * NC + c

    ds = _load_edges(src_hbm, dst_hbm, src_v, dst_v, wid, lsem)
    sl = pl.ds(s * NPT, NPT)
    dh = pltpu.async_copy(h1_hbm.at[sl], u_v, sem0)
    da0 = pltpu.async_copy(acc_hbm.at[0, sl], a0_v, sem0)
    da1 = pltpu.async_copy(acc_hbm.at[1, sl], a1_v, sem0)
    db = pltpu.async_copy(b1_hbm, b1_v, sem0)
    dwa = pltpu.async_copy(w2a_hbm, w2a_v, sem0)
    dwb = pltpu.async_copy(w2b_hbm, w2b_v, sem0)
    dd0 = pltpu.async_copy(degp_hbm.at[0, sl], d0_v, sem1)
    dd1 = pltpu.async_copy(degp_hbm.at[1, sl], d1_v, sem1)

    _zero_acc(rows_v, acc_sh, s, H)

    dd0.wait()
    dd1.wait()
    _mk_dinv(d0_v, d1_v)
    for d in (dh, da0, da1, db, dwa, dwb):
        d.wait()
    b1_vec = b1_v[...]

    def mk_u(i, _):
        dinv = plsc.load_gather(d0_v, [jnp.full((16,), i, jnp.int32)])
        h1s = u_v[i] * dinv
        out1 = (a0_v[i] + a1_v[i] + h1s) * dinv + b1_vec
        u_v[i] = jnp.maximum(out1, 0.0) * dinv
        return 0
    lax.fori_loop(0, NPT, mk_u, 0)
    pltpu.sync_copy(u_v, tab_hbm.at[c, pl.ds(s * NPT, NPT)])

    for d in ds:
        d.wait()
    plsc.subcore_barrier()

    _edge_pass(tab_hbm.at[c], src_v, dst_v, rows_v, acc_sh, [sem0, sem1])
    plsc.subcore_barrier()

    pltpu.sync_copy(acc_sh.at[pl.ds(s * NPT, NPT)], a0_v)
    pltpu.sync_copy(b2h_hbm, b1_v)
    flag = jnp.where(c == 0, 1.0, 0.0)
    w2a_vec = w2a_v[...]
    w2b_vec = w2b_v[...]
    b2_vec = b1_v[...]
    iota16 = lax.iota(jnp.int32, 16)

    def mk_q(g, _):
        ridx = iota16 + g * 16
        dinvg = d0_v[pl.ds(g * 16, 16)]
        qa = jnp.zeros((16,), jnp.float32)
        qb = jnp.zeros((16,), jnp.float32)
        for k in range(H):
            kidx = jnp.full((16,), k, jnp.int32)
            col = (plsc.load_gather(a0_v, [ridx, kidx])
                   + plsc.load_gather(u_v, [ridx, kidx]) * flag)
            qa = qa + col * w2a_vec[k]
            qb = qb + col * w2b_vec[k]
        qa_v[pl.ds(g * 16, 16)] = qa * dinvg + b2_vec[0]
        qb_v[pl.ds(g * 16, 16)] = qb * dinvg + b2_vec[1]
        return 0
    lax.fori_loop(0, NPT // 16, mk_q, 0)
    pltpu.sync_copy(qa_v, q_hbm.at[c, 0, pl.ds(s * NPT, NPT)])
    pltpu.sync_copy(qb_v, q_hbm.at[c, 1, pl.ds(s * NPT, NPT)])


@functools.partial(
    pl.kernel,
    out_type=jax.ShapeDtypeStruct((NP * C,), jnp.float32),
    mesh=_sc_mesh(),
    compiler_params=_SC_PARAMS,
    scratch_types=[
        pltpu.VMEM((NPW,), jnp.float32),
        pltpu.VMEM((NPW,), jnp.float32),
        pltpu.VMEM((NPW,), jnp.float32),
        pltpu.VMEM((NPW,), jnp.float32),
        pltpu.VMEM((NPW * C,), jnp.float32),
        pltpu.SemaphoreType.DMA,
    ],
)
def _final_kernel(q_hbm, out_hbm, qa_v, qb_v, t_v, t2_v, out_v, sem):
    c = lax.axis_index("c")
    s = lax.axis_index("s")
    wid = s * NC + c
    base = wid * NPW

    loads = [
        pltpu.async_copy(q_hbm.at[0, 0, pl.ds(base, NPW)], qa_v, sem),
        pltpu.async_copy(q_hbm.at[1, 0, pl.ds(base, NPW)], t_v, sem),
        pltpu.async_copy(q_hbm.at[0, 1, pl.ds(base, NPW)], qb_v, sem),
        pltpu.async_copy(q_hbm.at[1, 1, pl.ds(base, NPW)], t2_v, sem),
    ]
    for d in loads:
        d.wait()

    def add_ab(i, _):
        sl = pl.ds(i * 16, 16)
        qa_v[sl] = qa_v[sl] + t_v[sl]
        qb_v[sl] = qb_v[sl] + t2_v[sl]
        return 0
    lax.fori_loop(0, NPW // 16, add_ab, 0)

    iota16 = lax.iota(jnp.int32, 16)

    def lsm(g, _):
        sl = pl.ds(g * 16, 16)
        a = qa_v[sl]
        b = qb_v[sl]
        m = jnp.maximum(a, b)
        e = jnp.exp(jnp.minimum(a, b) - m)
        t = e / (2.0 + e)
        t2 = t * t
        ln = 2.0 * t * (1.0 + t2 * (1.0 / 3.0 + t2 * (0.2 + t2 * (
            1.0 / 7.0 + t2 * (1.0 / 9.0)))))
        oidx = iota16 * 2 + g * 32
        plsc.store_scatter(out_v, [oidx], a - m - ln)
        plsc.store_scatter(out_v, [oidx + 1], b - m - ln)
        return 0
    lax.fori_loop(0, NPW // 16, lsm, 0)
    pltpu.sync_copy(out_v, out_hbm.at[pl.ds(base * C, NPW * C)])


_EBLK = 65536


def _esplit_body(ei_ref, src_ref, dst_ref):
    src_ref[...] = ei_ref[0]
    dst_ref[...] = ei_ref[1]


def _edge_split(ei):
    return pl.pallas_call(
        _esplit_body,
        grid=(pl.cdiv(N_EDGES, _EBLK),),
        in_specs=[pl.BlockSpec((2, _EBLK), lambda i: (0, i))],
        out_specs=(
            pl.BlockSpec((_EBLK,), lambda i: (i,)),
            pl.BlockSpec((_EBLK,), lambda i: (i,)),
        ),
        out_shape=(
            jax.ShapeDtypeStruct((N_EDGES,), jnp.int32),
            jax.ShapeDtypeStruct((N_EDGES,), jnp.int32),
        ),
    )(ei)


_BLK = 512
_GRID = NP // _BLK


def _mm1_body(x_ref, w1_ref, out_ref):
    out_ref[...] = jnp.dot(x_ref[...], w1_ref[...],
                           preferred_element_type=jnp.float32)


def _mm1(x, W1):
    return pl.pallas_call(
        _mm1_body,
        grid=(_GRID,),
        in_specs=[
            pl.BlockSpec((_BLK, F_IN), lambda i: (i, 0)),
            pl.BlockSpec((F_IN, H), lambda i: (0, 0)),
        ],
        out_specs=pl.BlockSpec((_BLK, H), lambda i: (i, 0)),
        out_shape=jax.ShapeDtypeStruct((NP, H), jnp.float32),
    )(x, W1)


def kernel(x, edge_index, W1, b1, W2, b2):
    ei = edge_index.astype(jnp.int32)
    src, dst = _edge_split(ei)
    b1t = b1.astype(jnp.float32)
    w2a = W2[:, 0].astype(jnp.float32)
    w2b = W2[:, 1].astype(jnp.float32)
    b2h = jnp.tile(b2.astype(jnp.float32) * 0.5, 8)

    degp = _deg_kernel(dst)
    h1 = _mm1(x, W1)
    acc, h1s_tab = _agg1_kernel(src, dst, h1, degp)
    del h1s_tab
    q, u_tab = _agg2_kernel(src, dst, h1, degp, acc, b1t, w2a, w2b, b2h)
    del u_tab
    out = _final_kernel(q)
    return out[:N_NODES * C].reshape(N_NODES, C)
